# Initial kernel scaffold; baseline (speedup 1.0000x reference)
import jax
import jax.numpy as jnp
from jax.experimental import pallas as pl


def kernel(c, pos, neg, center_w, context_w):
    raise NotImplementedError("write your pallas kernel here")


# speedup vs baseline: 4.7851x; 4.7851x over previous
import functools

import jax
import jax.numpy as jnp
from jax import lax
from jax.experimental import pallas as pl
from jax.experimental.pallas import tpu as pltpu
from jax.experimental.pallas import tpu_sc as plsc

VOCAB = 1000000
EMBED = 64
B = 16384
NEG = 20

NC, NS, L = 2, 16, 16
NW = NC * NS
BPW = B // NW
CB = 64
NCHUNK = BPW // CB
NIDX_ROWS = CB * NEG // 128


def _sc_body(c_hbm, pos_hbm, neg_hbm, center_hbm, context_hbm,
             pos_out, neg_out,
             cidx, pidx, nidx, c_rows, p_rows, n_rows, pos_sv, neg_sv, sem):
    wid = lax.axis_index("s") * NC + lax.axis_index("c")
    lane = lax.iota(jnp.int32, L)
    last_lane = lane == (L - 1)

    def chunk_body(ch, carry):
        base = wid * BPW + ch * CB
        pltpu.sync_copy(c_hbm.at[pl.ds(base, CB)], cidx)
        pltpu.sync_copy(pos_hbm.at[pl.ds(base, CB)], pidx)
        pltpu.sync_copy(neg_hbm.at[pl.ds(base * NEG, CB * NEG)], nidx)
        cps = [pltpu.async_copy(center_hbm.at[cidx], c_rows, sem),
               pltpu.async_copy(context_hbm.at[pidx], p_rows, sem)]
        for j in range(NIDX_ROWS):
            cps.append(pltpu.async_copy(
                context_hbm.at[nidx.at[pl.ds(j * 128, 128)]],
                n_rows.at[pl.ds(j * 128, 128), :], sem))
        for cp in cps:
            cp.wait()

        def elem_body(i, carry2):
            cvs = tuple(c_rows[i, pl.ds(db * L, L)] for db in range(EMBED // L))
            pvs = tuple(p_rows[i, pl.ds(db * L, L)] for db in range(EMBED // L))
            s = cvs[0] * pvs[0]
            for db in range(1, EMBED // L):
                s = s + cvs[db] * pvs[db]
            plsc.store_scatter(pos_sv, [jnp.full((L,), i, jnp.int32)],
                               plsc.cumsum(s), mask=last_lane)
            for k in range(NEG):
                r = i * NEG + k
                s2 = n_rows[r, pl.ds(0, L)] * cvs[0]
                for db in range(1, EMBED // L):
                    s2 = s2 + n_rows[r, pl.ds(db * L, L)] * cvs[db]
                plsc.store_scatter(neg_sv, [jnp.full((L,), r, jnp.int32)],
                                   plsc.cumsum(s2), mask=last_lane)
            return carry2

        lax.fori_loop(0, CB, elem_body, 0)
        pltpu.sync_copy(pos_sv, pos_out.at[pl.ds(base, CB)])
        pltpu.sync_copy(neg_sv, neg_out.at[pl.ds(base * NEG, CB * NEG)])
        return carry

    lax.fori_loop(0, NCHUNK, chunk_body, 0)


@functools.lru_cache(maxsize=None)
def _build_sc_scores():
  return functools.partial(
    pl.kernel,
    out_type=(jax.ShapeDtypeStruct((B,), jnp.float32),
              jax.ShapeDtypeStruct((B * NEG,), jnp.float32)),
    mesh=plsc.VectorSubcoreMesh(core_axis_name="c", subcore_axis_name="s",
                                num_cores=NC, num_subcores=NS),
    compiler_params=pltpu.CompilerParams(needs_layout_passes=False,
                                         use_tc_tiling_on_sc=False),
    scratch_types=[
        pltpu.VMEM((CB,), jnp.int32),
        pltpu.VMEM((CB,), jnp.int32),
        pltpu.VMEM((CB * NEG,), jnp.int32),
        pltpu.VMEM((CB, EMBED), jnp.float32),
        pltpu.VMEM((CB, EMBED), jnp.float32),
        pltpu.VMEM((CB * NEG, EMBED), jnp.float32),
        pltpu.VMEM((CB,), jnp.float32),
        pltpu.VMEM((CB * NEG,), jnp.float32),
        pltpu.SemaphoreType.DMA,
    ],
  )(_sc_body)


def _loss_body(pos_ref, neg_ref, out_ref):
    eps = 1e-07
    ps = pos_ref[...]
    ns = neg_ref[...]
    pos_loss = -jnp.log(jax.nn.sigmoid(ps) + eps)
    neg_loss = -jnp.log(jax.nn.sigmoid(-ns) + eps)
    out_ref[0, 0] = (jnp.sum(pos_loss) / float(B)
                     + jnp.sum(neg_loss) / float(B * NEG))


_tc_loss = pl.pallas_call(
    _loss_body,
    out_shape=jax.ShapeDtypeStruct((1, 1), jnp.float32),
    out_specs=pl.BlockSpec(memory_space=pltpu.SMEM),
)


def kernel(c, pos, neg, center_w, context_w):
    c = c.astype(jnp.int32)
    pos = pos.astype(jnp.int32)
    negf = neg.astype(jnp.int32).reshape(B * NEG)
    pos_s, neg_s = _build_sc_scores()(c, pos, negf, center_w, context_w)
    loss = _tc_loss(pos_s.reshape(B // 128, 128),
                    neg_s.reshape(B * NEG // 128, 128))
    return loss[0, 0]


# speedup vs baseline: 5.9229x; 1.2378x over previous
import functools

import jax
import jax.numpy as jnp
from jax import lax
from jax.experimental import pallas as pl
from jax.experimental.pallas import tpu as pltpu
from jax.experimental.pallas import tpu_sc as plsc

VOCAB = 1000000
EMBED = 64
B = 16384
NEG = 20

NC, NS, L = 2, 16, 16
NW = NC * NS
BPW = B // NW
CB = 64
NCHUNK = BPW // CB
NIDX_ROWS = CB * NEG // 128


def _sc_body(cemb_hbm, pos_hbm, neg_hbm, context_hbm,
             pos_out, neg_out,
             pidx, nidx, c_rows, p_rows, n_rows, pos_sv, neg_sv, sem):
    wid = lax.axis_index("s") * NC + lax.axis_index("c")
    lane = lax.iota(jnp.int32, L)
    last_lane = lane == (L - 1)

    def chunk_body(ch, carry):
        base = wid * BPW + ch * CB
        pltpu.sync_copy(pos_hbm.at[pl.ds(base, CB)], pidx)
        pltpu.sync_copy(neg_hbm.at[pl.ds(base * NEG, CB * NEG)], nidx)
        cps = [pltpu.async_copy(cemb_hbm.at[pl.ds(base, CB), :], c_rows, sem),
               pltpu.async_copy(context_hbm.at[pidx], p_rows, sem)]
        for j in range(NIDX_ROWS):
            cps.append(pltpu.async_copy(
                context_hbm.at[nidx.at[pl.ds(j * 128, 128)]],
                n_rows.at[pl.ds(j * 128, 128), :], sem))
        for cp in cps:
            cp.wait()

        def elem_body(i, carry2):
            cvs = tuple(c_rows[i, pl.ds(db * L, L)] for db in range(EMBED // L))
            pvs = tuple(p_rows[i, pl.ds(db * L, L)] for db in range(EMBED // L))
            s = cvs[0] * pvs[0]
            for db in range(1, EMBED // L):
                s = s + cvs[db] * pvs[db]
            plsc.store_scatter(pos_sv, [jnp.full((L,), i, jnp.int32)],
                               plsc.cumsum(s), mask=last_lane)
            for k in range(NEG):
                r = i * NEG + k
                s2 = n_rows[r, pl.ds(0, L)] * cvs[0]
                for db in range(1, EMBED // L):
                    s2 = s2 + n_rows[r, pl.ds(db * L, L)] * cvs[db]
                plsc.store_scatter(neg_sv, [jnp.full((L,), r, jnp.int32)],
                                   plsc.cumsum(s2), mask=last_lane)
            return carry2

        lax.fori_loop(0, CB, elem_body, 0)
        pltpu.sync_copy(pos_sv, pos_out.at[pl.ds(base, CB)])
        pltpu.sync_copy(neg_sv, neg_out.at[pl.ds(base * NEG, CB * NEG)])
        return carry

    lax.fori_loop(0, NCHUNK, chunk_body, 0)


@functools.lru_cache(maxsize=None)
def _build_sc_scores():
  return functools.partial(
    pl.kernel,
    out_type=(jax.ShapeDtypeStruct((B,), jnp.float32),
              jax.ShapeDtypeStruct((B * NEG,), jnp.float32)),
    mesh=plsc.VectorSubcoreMesh(core_axis_name="c", subcore_axis_name="s",
                                num_cores=NC, num_subcores=NS),
    compiler_params=pltpu.CompilerParams(needs_layout_passes=False,
                                         use_tc_tiling_on_sc=False),
    scratch_types=[
        pltpu.VMEM((CB,), jnp.int32),
        pltpu.VMEM((CB * NEG,), jnp.int32),
        pltpu.VMEM((CB, EMBED), jnp.float32),
        pltpu.VMEM((CB, EMBED), jnp.float32),
        pltpu.VMEM((CB * NEG, EMBED), jnp.float32),
        pltpu.VMEM((CB,), jnp.float32),
        pltpu.VMEM((CB * NEG,), jnp.float32),
        pltpu.SemaphoreType.DMA,
    ],
  )(_sc_body)


def _loss_body(pos_ref, neg_ref, out_ref):
    eps = 1e-07
    ps = pos_ref[...]
    ns = neg_ref[...]
    pos_loss = -jnp.log(jax.nn.sigmoid(ps) + eps)
    neg_loss = -jnp.log(jax.nn.sigmoid(-ns) + eps)
    out_ref[0, 0] = (jnp.sum(pos_loss) / float(B)
                     + jnp.sum(neg_loss) / float(B * NEG))


_tc_loss = pl.pallas_call(
    _loss_body,
    out_shape=jax.ShapeDtypeStruct((1, 1), jnp.float32),
    out_specs=pl.BlockSpec(memory_space=pltpu.SMEM),
)


def kernel(c, pos, neg, center_w, context_w):
    c = c.astype(jnp.int32)
    pos = pos.astype(jnp.int32)
    negf = neg.astype(jnp.int32).reshape(B * NEG)
    c_emb = jnp.take(center_w, c, axis=0)
    pos_s, neg_s = _build_sc_scores()(c_emb, pos, negf, context_w)
    loss = _tc_loss(pos_s.reshape(B // 128, 128),
                    neg_s.reshape(B * NEG // 128, 128))
    return loss[0, 0]
